# Initial kernel scaffold; baseline (speedup 1.0000x reference)
import jax
import jax.numpy as jnp
from jax.experimental import pallas as pl


def kernel(sequence_emb, mask, pos_table, W1_imp, b1_imp, W2_imp, b2_imp, W1_enc, b1_enc, W2_enc, b2_enc):
    raise NotImplementedError("write your pallas kernel here")


# speedup vs baseline: 1.6117x; 1.6117x over previous
import functools

import jax
import jax.numpy as jnp
from jax.experimental import pallas as pl
from jax.experimental.pallas import tpu as pltpu

B, L, D = 1024, 200, 128
H, O = 512, 128
TOP_K = 50
TB = 8


def _pool_kernel(seq_ref, mask_ref, pos_ref, w1i_ref, b1i_ref, w2i_ref,
                 w1e_ref, b1e_ref, w2e_ref, b2e_ref, out_ref,
                 p_imp_scr, p_enc_scr):
    step = pl.program_id(0)

    @pl.when(step == 0)
    def _():
        pos = pos_ref[...]
        p_imp_scr[...] = (
            jnp.dot(pos, w1i_ref[D:, :], preferred_element_type=jnp.float32)
            + b1i_ref[...])
        p_enc_scr[...] = (
            jnp.dot(pos, w1e_ref[D:, :], preferred_element_type=jnp.float32)
            + b1e_ref[...])

    seq = seq_ref[...]
    seq2d = seq.reshape(TB * L, D)

    a = jnp.dot(seq2d, w1i_ref[:D, :], preferred_element_type=jnp.float32)
    h = jnp.maximum(a.reshape(TB, L, H) + p_imp_scr[...][None, :, :], 0.0)
    scores = jnp.sum(h * w2i_ref[0, :][None, None, :], axis=-1)
    scores = jnp.where(mask_ref[...] == 0, jnp.float32(-1e9), scores)

    s_i = scores[:, :, None]
    s_j = scores[:, None, :]
    ii = jax.lax.broadcasted_iota(jnp.int32, (TB, L, L), 1)
    jj = jax.lax.broadcasted_iota(jnp.int32, (TB, L, L), 2)
    beats = (s_j > s_i) | ((s_j == s_i) & (jj < ii))
    rank = jnp.sum(jnp.where(beats, 1.0, 0.0), axis=2)
    selw = (rank < TOP_K).astype(jnp.float32)

    a2 = jnp.dot(seq2d, w1e_ref[:D, :], preferred_element_type=jnp.float32)
    h2 = jnp.maximum(a2.reshape(TB, L, H) + p_enc_scr[...][None, :, :], 0.0)
    pooled = jnp.sum(h2 * selw[:, :, None], axis=1) * (1.0 / TOP_K)
    out_ref[...] = (
        jnp.dot(pooled, w2e_ref[...], preferred_element_type=jnp.float32)
        + b2e_ref[...])


@jax.jit
def kernel(sequence_emb, mask, pos_table, W1_imp, b1_imp, W2_imp, b2_imp,
           W1_enc, b1_enc, W2_enc, b2_enc):
    del b2_imp
    pos = pos_table[:L]
    b1i = b1_imp.reshape(1, H)
    b1e = b1_enc.reshape(1, H)
    w2i = W2_imp.reshape(1, H)
    b2e = b2_enc.reshape(1, O)

    grid = (B // TB,)
    out = pl.pallas_call(
        _pool_kernel,
        grid=grid,
        in_specs=[
            pl.BlockSpec((TB, L, D), lambda i: (i, 0, 0)),
            pl.BlockSpec((TB, L), lambda i: (i, 0)),
            pl.BlockSpec((L, D), lambda i: (0, 0)),
            pl.BlockSpec((2 * D, H), lambda i: (0, 0)),
            pl.BlockSpec((1, H), lambda i: (0, 0)),
            pl.BlockSpec((1, H), lambda i: (0, 0)),
            pl.BlockSpec((2 * D, H), lambda i: (0, 0)),
            pl.BlockSpec((1, H), lambda i: (0, 0)),
            pl.BlockSpec((H, O), lambda i: (0, 0)),
            pl.BlockSpec((1, O), lambda i: (0, 0)),
        ],
        out_specs=pl.BlockSpec((TB, O), lambda i: (i, 0)),
        out_shape=jax.ShapeDtypeStruct((B, O), jnp.float32),
        scratch_shapes=[
            pltpu.VMEM((L, H), jnp.float32),
            pltpu.VMEM((L, H), jnp.float32),
        ],
        compiler_params=pltpu.CompilerParams(
            dimension_semantics=("arbitrary",),
        ),
    )(sequence_emb, mask, pos, W1_imp, b1i, w2i, W1_enc, b1e, W2_enc, b2e)
    return out
